# baseline (speedup 1.0000x reference)
import functools

import jax
import jax.numpy as jnp
from jax import lax
from jax.experimental import pallas as pl
from jax.experimental.pallas import tpu as pltpu
from jax.experimental.pallas import tpu_sc as plsc

GRP = 640
NBUF = 2


@functools.cache
def _make_gather(n: int, d: int):
    info = plsc.get_sparse_core_info()
    nw = info.num_cores * info.num_subcores
    assert n % (nw * GRP) == 0
    rpw = n // nw
    ngrp = rpw // GRP

    mesh = plsc.VectorSubcoreMesh(core_axis_name="c", subcore_axis_name="s")

    @functools.partial(
        pl.kernel,
        mesh=mesh,
        out_type=jax.ShapeDtypeStruct((n, d), jnp.float32),
        scratch_types=[
            pltpu.VMEM((rpw,), jnp.int32),
            pltpu.VMEM((NBUF, GRP, d), jnp.float32),
        ]
        + [pltpu.SemaphoreType.DMA] * (2 * NBUF),
        compiler_params=pltpu.CompilerParams(use_tc_tiling_on_sc=False),
    )
    def gather_kernel(table_hbm, idx_hbm, out_hbm, idx_v, rows_v, *sems):
        gsem = sems[:NBUF]
        ssem = sems[NBUF:]
        wid = lax.axis_index("s") * info.num_cores + lax.axis_index("c")
        row0 = wid * rpw

        pltpu.sync_copy(idx_hbm.at[pl.ds(row0, rpw)], idx_v)

        def gather_desc(g, b, make):
            return make(
                table_hbm.at[idx_v.at[pl.ds(g * GRP, GRP)]], rows_v.at[b], gsem[b]
            )

        def store_desc(g, b, make):
            return make(rows_v.at[b], out_hbm.at[pl.ds(row0 + g * GRP, GRP)], ssem[b])

        gather_desc(0, 0, pltpu.async_copy)

        def body(gp, carry):
            for buf in range(2):
                g = gp * 2 + buf
                nbuf = 1 - buf
                gather_desc(g, buf, pltpu.make_async_copy).wait()

                @pl.when(g + 1 < ngrp)
                def _():
                    @pl.when(g >= 1)
                    def _():
                        store_desc(g - 1, nbuf, pltpu.make_async_copy).wait()

                    gather_desc(g + 1, nbuf, pltpu.async_copy)

                store_desc(g, buf, pltpu.async_copy)
            return carry

        assert ngrp % 2 == 0
        lax.fori_loop(0, ngrp // 2, body, 0)

        store_desc(ngrp - 1, (ngrp - 1) % 2, pltpu.make_async_copy).wait()
        store_desc(ngrp - 2, (ngrp - 2) % 2, pltpu.make_async_copy).wait()

    return gather_kernel


VB = 256
HOFF = 499968


@functools.cache
def _make_tc_repack(v: int, d: int):
    npk = v - HOFF
    assert npk >= v - npk and HOFF % VB == 0
    nb = -(-npk // VB)
    nboff = HOFF // VB

    def body(xl_ref, xr_ref, o_ref):
        ident = (
            lax.broadcasted_iota(jnp.int32, (d, d), 0)
            == lax.broadcasted_iota(jnp.int32, (d, d), 1)
        ).astype(jnp.float32)
        o_ref[:, :d] = lax.dot_general(
            xl_ref[...], ident, (((0,), (0,)), ((), ())),
            precision=lax.Precision.HIGHEST,
            preferred_element_type=jnp.float32,
        )
        o_ref[:, d:] = lax.dot_general(
            xr_ref[...], ident, (((0,), (0,)), ((), ())),
            precision=lax.Precision.HIGHEST,
            preferred_element_type=jnp.float32,
        )

    return pl.pallas_call(
        body,
        grid=(nb,),
        in_specs=[
            pl.BlockSpec((d, VB), lambda i: (0, i)),
            pl.BlockSpec((d, VB), lambda i: (0, i + nboff)),
        ],
        out_specs=pl.BlockSpec((VB, 2 * d), lambda i: (i, 0)),
        out_shape=jax.ShapeDtypeStruct((npk, 2 * d), jnp.float32),
    )


def kernel(src, src_attn_mask, embedding_table):
    b, s = src.shape
    v, d = embedding_table.shape
    tableT = embedding_table.T
    packed = _make_tc_repack(v, d)(tableT, tableT)
    npk = packed.shape[0]
    idx = src.reshape(-1).astype(jnp.int32)
    idx2 = jnp.where(idx < npk, 2 * idx, 2 * (idx - HOFF) + 1)
    out = _make_gather(idx.shape[0], d)(packed.reshape(2 * npk, d), idx2)
    return out.reshape(b, s, d)


# speedup vs baseline: 1.7965x; 1.7965x over previous
import functools

import jax
import jax.numpy as jnp
from jax import lax
from jax.experimental import pallas as pl
from jax.experimental.pallas import tpu as pltpu
from jax.experimental.pallas import tpu_sc as plsc

GRP = 640
NBUF = 2


@functools.cache
def _make_gather(n: int, d: int):
    info = plsc.get_sparse_core_info()
    nw = info.num_cores * info.num_subcores
    assert n % (nw * GRP) == 0
    rpw = n // nw
    ngrp = rpw // GRP

    mesh = plsc.VectorSubcoreMesh(core_axis_name="c", subcore_axis_name="s")

    @functools.partial(
        pl.kernel,
        mesh=mesh,
        out_type=jax.ShapeDtypeStruct((n, d), jnp.float32),
        scratch_types=[
            pltpu.VMEM((rpw,), jnp.int32),
            pltpu.VMEM((NBUF, GRP, d), jnp.float32),
        ]
        + [pltpu.SemaphoreType.DMA] * (2 * NBUF),
        compiler_params=pltpu.CompilerParams(use_tc_tiling_on_sc=False),
    )
    def gather_kernel(table_hbm, idx_hbm, out_hbm, idx_v, rows_v, *sems):
        gsem = sems[:NBUF]
        ssem = sems[NBUF:]
        wid = lax.axis_index("s") * info.num_cores + lax.axis_index("c")
        row0 = wid * rpw

        pltpu.sync_copy(idx_hbm.at[pl.ds(row0, rpw)], idx_v)

        def gather_desc(g, b, make):
            return make(
                table_hbm.at[idx_v.at[pl.ds(g * GRP, GRP)]], rows_v.at[b], gsem[b]
            )

        def store_desc(g, b, make):
            return make(rows_v.at[b], out_hbm.at[pl.ds(row0 + g * GRP, GRP)], ssem[b])

        gather_desc(0, 0, pltpu.async_copy)

        def body(gp, carry):
            for buf in range(2):
                g = gp * 2 + buf
                nbuf = 1 - buf
                gather_desc(g, buf, pltpu.make_async_copy).wait()

                @pl.when(g + 1 < ngrp)
                def _():
                    @pl.when(g >= 1)
                    def _():
                        store_desc(g - 1, nbuf, pltpu.make_async_copy).wait()

                    gather_desc(g + 1, nbuf, pltpu.async_copy)

                store_desc(g, buf, pltpu.async_copy)
            return carry

        assert ngrp % 2 == 0
        lax.fori_loop(0, ngrp // 2, body, 0)

        store_desc(ngrp - 1, (ngrp - 1) % 2, pltpu.make_async_copy).wait()
        store_desc(ngrp - 2, (ngrp - 2) % 2, pltpu.make_async_copy).wait()

    return gather_kernel


VB = 4096
HOFF = 499712


@functools.cache
def _make_tc_repack(v: int, d: int):
    npk = v - HOFF
    assert npk >= v - npk and HOFF % VB == 0
    nb = -(-npk // VB)
    nboff = HOFF // VB

    def body(xl_ref, xr_ref, o_ref):
        ident = (
            lax.broadcasted_iota(jnp.int32, (d, d), 0)
            == lax.broadcasted_iota(jnp.int32, (d, d), 1)
        ).astype(jnp.float32)
        o_ref[:, :d] = lax.dot_general(
            xl_ref[...], ident, (((0,), (0,)), ((), ())),
            precision=lax.Precision.HIGHEST,
            preferred_element_type=jnp.float32,
        )
        o_ref[:, d:] = lax.dot_general(
            xr_ref[...], ident, (((0,), (0,)), ((), ())),
            precision=lax.Precision.HIGHEST,
            preferred_element_type=jnp.float32,
        )

    return pl.pallas_call(
        body,
        grid=(nb,),
        in_specs=[
            pl.BlockSpec((d, VB), lambda i: (0, i)),
            pl.BlockSpec((d, VB), lambda i: (0, i + nboff)),
        ],
        out_specs=pl.BlockSpec((VB, 2 * d), lambda i: (i, 0)),
        out_shape=jax.ShapeDtypeStruct((npk, 2 * d), jnp.float32),
    )


def kernel(src, src_attn_mask, embedding_table):
    b, s = src.shape
    v, d = embedding_table.shape
    tableT = embedding_table.T
    packed = _make_tc_repack(v, d)(tableT, tableT)
    npk = packed.shape[0]
    idx = src.reshape(-1).astype(jnp.int32)
    idx2 = jnp.where(idx < npk, 2 * idx, 2 * (idx - HOFF) + 1)
    out = _make_gather(idx.shape[0], d)(packed.reshape(2 * npk, d), idx2)
    return out.reshape(b, s, d)


# speedup vs baseline: 1.8243x; 1.0154x over previous
import functools

import jax
import jax.numpy as jnp
from jax import lax
from jax.experimental import pallas as pl
from jax.experimental.pallas import tpu as pltpu
from jax.experimental.pallas import tpu_sc as plsc

GRP = 640
NBUF = 2


@functools.cache
def _make_gather(n: int, d: int):
    info = plsc.get_sparse_core_info()
    nw = info.num_cores * info.num_subcores
    assert n % (nw * GRP) == 0
    rpw = n // nw
    ngrp = rpw // GRP

    mesh = plsc.VectorSubcoreMesh(core_axis_name="c", subcore_axis_name="s")

    @functools.partial(
        pl.kernel,
        mesh=mesh,
        out_type=jax.ShapeDtypeStruct((n, d), jnp.float32),
        scratch_types=[
            pltpu.VMEM((rpw,), jnp.int32),
            pltpu.VMEM((NBUF, GRP, d), jnp.float32),
        ]
        + [pltpu.SemaphoreType.DMA] * (2 * NBUF),
        compiler_params=pltpu.CompilerParams(use_tc_tiling_on_sc=False),
    )
    def gather_kernel(table_hbm, idx_hbm, out_hbm, idx_v, rows_v, *sems):
        gsem = sems[:NBUF]
        ssem = sems[NBUF:]
        wid = lax.axis_index("s") * info.num_cores + lax.axis_index("c")
        row0 = wid * rpw

        pltpu.sync_copy(idx_hbm.at[pl.ds(row0, rpw)], idx_v)

        def gather_desc(g, b, make):
            return make(
                table_hbm.at[idx_v.at[pl.ds(g * GRP, GRP)]], rows_v.at[b], gsem[b]
            )

        def store_desc(g, b, make):
            return make(rows_v.at[b], out_hbm.at[pl.ds(row0 + g * GRP, GRP)], ssem[b])

        gather_desc(0, 0, pltpu.async_copy)

        def body(gp, carry):
            for buf in range(2):
                g = gp * 2 + buf
                nbuf = 1 - buf
                gather_desc(g, buf, pltpu.make_async_copy).wait()

                @pl.when(g + 1 < ngrp)
                def _():
                    @pl.when(g >= 1)
                    def _():
                        store_desc(g - 1, nbuf, pltpu.make_async_copy).wait()

                    gather_desc(g + 1, nbuf, pltpu.async_copy)

                store_desc(g, buf, pltpu.async_copy)
            return carry

        assert ngrp % 2 == 0
        lax.fori_loop(0, ngrp // 2, body, 0)

        store_desc(ngrp - 1, (ngrp - 1) % 2, pltpu.make_async_copy).wait()
        store_desc(ngrp - 2, (ngrp - 2) % 2, pltpu.make_async_copy).wait()

    return gather_kernel


VB = 8192
HOFF = 499712


@functools.cache
def _make_tc_repack(v: int, d: int):
    npk = v - HOFF
    assert npk >= v - npk and HOFF % VB == 0
    nb = -(-npk // VB)
    nboff = HOFF // VB

    def body(xl_ref, xr_ref, o_ref):
        ident = (
            lax.broadcasted_iota(jnp.int32, (d, d), 0)
            == lax.broadcasted_iota(jnp.int32, (d, d), 1)
        ).astype(jnp.float32)
        o_ref[:, :d] = lax.dot_general(
            xl_ref[...], ident, (((0,), (0,)), ((), ())),
            precision=lax.Precision.HIGHEST,
            preferred_element_type=jnp.float32,
        )
        o_ref[:, d:] = lax.dot_general(
            xr_ref[...], ident, (((0,), (0,)), ((), ())),
            precision=lax.Precision.HIGHEST,
            preferred_element_type=jnp.float32,
        )

    return pl.pallas_call(
        body,
        grid=(nb,),
        in_specs=[
            pl.BlockSpec((d, VB), lambda i: (0, i)),
            pl.BlockSpec((d, VB), lambda i: (0, i + nboff)),
        ],
        out_specs=pl.BlockSpec((VB, 2 * d), lambda i: (i, 0)),
        out_shape=jax.ShapeDtypeStruct((npk, 2 * d), jnp.float32),
    )


def kernel(src, src_attn_mask, embedding_table):
    b, s = src.shape
    v, d = embedding_table.shape
    tableT = embedding_table.T
    packed = _make_tc_repack(v, d)(tableT, tableT)
    npk = packed.shape[0]
    idx = src.reshape(-1).astype(jnp.int32)
    idx2 = jnp.where(idx < npk, 2 * idx, 2 * (idx - HOFF) + 1)
    out = _make_gather(idx.shape[0], d)(packed.reshape(2 * npk, d), idx2)
    return out.reshape(b, s, d)
